# baseline (speedup 1.0000x reference)
import functools

import jax
import jax.numpy as jnp
import numpy as np
from jax import lax
from jax.experimental import pallas as pl
from jax.experimental.pallas import tpu as pltpu
from jax.experimental.pallas import tpu_sc as plsc

N = 10000
E = 320000
D = 128
NC = 2
NS = 16
L = 16
N_TAB = 10112
ROWS_PER_TILE = N_TAB // NS
C = 128
E_PAD = 327680
EPT_CONV = E_PAD // NS
EPT_DEG = E_PAD // (NC * NS)
NBUF = 2
NCHUNK = EPT_CONV // C

f32 = jnp.float32
i32 = jnp.int32

_CONSTS = None


def _consts():
    global _CONSTS
    if _CONSTS is None:
        rowvalid = np.zeros((N_TAB, 1), np.float32)
        rowvalid[:N] = 1.0
        zeros1 = np.zeros((N_TAB,), np.float32)
        _CONSTS = (rowvalid, zeros1)
    return _CONSTS


def _rng_consts():
    mask = jax.random.bernoulli(jax.random.key(42), 1.0 - 0.1, (N, D))
    maskscale = jnp.where(mask, jnp.float32(1.0 / 0.9), jnp.float32(0.0))
    maskscale = jnp.concatenate([maskscale, jnp.zeros((N_TAB - N, D), f32)])
    eps = jax.random.normal(jax.random.key(43), (N, D), dtype=f32)
    eps = jnp.concatenate([eps, jnp.zeros((N_TAB - N, D), f32)])
    return maskscale, eps


def _degrees_body(dst_hbm, ew_hbm, outw0, outc0, outw1, outc1,
                  accw_sh, accc_sh, dst_v, ew_v, ones_v, stage_v, sem):
    c = lax.axis_index("c")
    s = lax.axis_index("s")
    nbase = s * ROWS_PER_TILE
    zer = jnp.zeros((L,), f32)
    for grp in range(640 // L):
        stage_v[pl.ds(grp * L, L)] = zer
    pltpu.sync_copy(stage_v.at[pl.ds(0, ROWS_PER_TILE)],
                    accw_sh.at[pl.ds(nbase, ROWS_PER_TILE)])
    pltpu.sync_copy(stage_v.at[pl.ds(0, ROWS_PER_TILE)],
                    accc_sh.at[pl.ds(nbase, ROWS_PER_TILE)])
    one = jnp.ones((L,), f32)
    for grp in range(C // L):
        ones_v[pl.ds(grp * L, L)] = one
    plsc.subcore_barrier()

    ebase = (c * NS + s) * EPT_DEG

    def chunk(g, carry):
        off = ebase + g * C
        pltpu.sync_copy(dst_hbm.at[pl.ds(off, C)], dst_v)
        pltpu.sync_copy(ew_hbm.at[pl.ds(off, C)], ew_v)
        pltpu.sync_copy(ew_v, accw_sh.at[dst_v], add=True)
        pltpu.sync_copy(ones_v, accc_sh.at[dst_v], add=True)
        return carry

    lax.fori_loop(0, EPT_DEG // C, chunk, 0)
    plsc.subcore_barrier()

    def writeout(acc_sh, out_hbm):
        pltpu.sync_copy(acc_sh.at[pl.ds(nbase, ROWS_PER_TILE)],
                        stage_v.at[pl.ds(0, ROWS_PER_TILE)])
        pltpu.sync_copy(stage_v.at[pl.ds(0, ROWS_PER_TILE)],
                        out_hbm.at[pl.ds(nbase, ROWS_PER_TILE)])

    @pl.when(c == 0)
    def _():
        writeout(accw_sh, outw0)
        writeout(accc_sh, outc0)

    @pl.when(c == 1)
    def _():
        writeout(accw_sh, outw1)
        writeout(accc_sh, outc1)


_degrees_call = pl.kernel(
    _degrees_body,
    out_type=(jax.ShapeDtypeStruct((N_TAB,), f32),
              jax.ShapeDtypeStruct((N_TAB,), f32),
              jax.ShapeDtypeStruct((N_TAB,), f32),
              jax.ShapeDtypeStruct((N_TAB,), f32)),
    mesh=plsc.VectorSubcoreMesh(core_axis_name="c", subcore_axis_name="s"),
    scratch_types=[
        pltpu.VMEM_SHARED((N_TAB,), f32),
        pltpu.VMEM_SHARED((N_TAB,), f32),
        pltpu.VMEM((C,), i32),
        pltpu.VMEM((C,), f32),
        pltpu.VMEM((C,), f32),
        pltpu.VMEM((640,), f32),
        pltpu.SemaphoreType.DMA,
    ],
)


def _make_dualconv(weighted):
    def body(tab0_hbm, tab1_hbm, src_hbm, dst_hbm, *rest):
        if weighted:
            ew_hbm = rest[0]
            rest = rest[1:]
        else:
            ew_hbm = None
        (out0, out1, acc_sh, src_v, dst_v, ew_v, rows_v) = rest[:7]
        sem_i = rest[7:7 + NBUF]
        sem_g = rest[7 + NBUF:7 + 2 * NBUF]
        c = lax.axis_index("c")
        s = lax.axis_index("s")
        nbase = s * ROWS_PER_TILE

        @pl.when(c == 0)
        def _():
            pltpu.sync_copy(tab0_hbm.at[pl.ds(nbase, ROWS_PER_TILE)],
                            acc_sh.at[pl.ds(nbase, ROWS_PER_TILE)])

        @pl.when(c == 1)
        def _():
            pltpu.sync_copy(tab1_hbm.at[pl.ds(nbase, ROWS_PER_TILE)],
                            acc_sh.at[pl.ds(nbase, ROWS_PER_TILE)])

        plsc.subcore_barrier()
        ebase = s * EPT_CONV

        def issue_idx(g, b):
            off = ebase + g * C
            pltpu.async_copy(src_hbm.at[pl.ds(off, C)], src_v.at[b], sem_i[b])
            pltpu.async_copy(dst_hbm.at[pl.ds(off, C)], dst_v.at[b], sem_i[b])
            if weighted:
                @pl.when(c == 0)
                def _():
                    pltpu.async_copy(ew_hbm.at[pl.ds(off, C)], ew_v.at[b],
                                     sem_i[b])

        def wait_idx(b):
            pltpu.make_async_copy(src_hbm.at[pl.ds(0, C)], src_v.at[b],
                                  sem_i[b]).wait()
            pltpu.make_async_copy(dst_hbm.at[pl.ds(0, C)], dst_v.at[b],
                                  sem_i[b]).wait()
            if weighted:
                @pl.when(c == 0)
                def _():
                    pltpu.make_async_copy(src_hbm.at[pl.ds(0, C)],
                                          ew_v.at[b], sem_i[b]).wait()

        def issue_gather(b):
            @pl.when(c == 0)
            def _():
                pltpu.async_copy(tab0_hbm.at[src_v.at[b]], rows_v.at[b],
                                 sem_g[b])

            @pl.when(c == 1)
            def _():
                pltpu.async_copy(tab1_hbm.at[src_v.at[b]], rows_v.at[b],
                                 sem_g[b])

        def wait_gather(b):
            pltpu.make_async_copy(tab0_hbm.at[pl.ds(0, C)], rows_v.at[b],
                                  sem_g[b]).wait()

        def mult_scatter(b):
            if weighted:
                @pl.when(c == 0)
                def _():
                    rb = rows_v.at[b]
                    eb = ew_v.at[b]

                    def mult_group(i, cc):
                        ewv = eb[pl.ds(i * L, L)]
                        for l in range(L):
                            sv = ewv.at[jnp.full((L,), l, i32)].get(
                                mode="promise_in_bounds")
                            e = i * L + l
                            for j in range(D // L):
                                rb[e, pl.ds(j * L, L)] = (
                                    rb[e, pl.ds(j * L, L)] * sv)
                        return cc

                    lax.fori_loop(0, C // L, mult_group, 0)
            pltpu.sync_copy(rows_v.at[b], acc_sh.at[dst_v.at[b]], add=True)

        for b in range(NBUF):
            issue_idx(b, b)
        wait_idx(0)
        issue_gather(0)

        def block(g0, carry):
            for b in range(NBUF):
                g = g0 * NBUF + b
                wait_gather(b)
                mult_scatter(b)
                issue_idx(g + NBUF, b)
                wait_idx((b + 1) % NBUF)
                issue_gather((b + 1) % NBUF)
            return carry

        lax.fori_loop(0, NCHUNK // NBUF - 1, block, 0)
        for gg in range(NCHUNK - NBUF, NCHUNK):
            b = gg % NBUF
            wait_gather(b)
            mult_scatter(b)
            if gg + 1 < NCHUNK:
                wait_idx((b + 1) % NBUF)
                issue_gather((b + 1) % NBUF)
        plsc.subcore_barrier()

        @pl.when(c == 0)
        def _():
            pltpu.sync_copy(acc_sh.at[pl.ds(nbase, ROWS_PER_TILE)],
                            out0.at[pl.ds(nbase, ROWS_PER_TILE)])

        @pl.when(c == 1)
        def _():
            pltpu.sync_copy(acc_sh.at[pl.ds(nbase, ROWS_PER_TILE)],
                            out1.at[pl.ds(nbase, ROWS_PER_TILE)])

    scratch = [
        pltpu.VMEM_SHARED((N_TAB, D), f32),
        pltpu.VMEM((NBUF, C), i32),
        pltpu.VMEM((NBUF, C), i32),
        pltpu.VMEM((NBUF, C), f32),
        pltpu.VMEM((NBUF, C, D), f32),
    ] + [pltpu.SemaphoreType.DMA] * (2 * NBUF)
    return pl.kernel(
        body,
        out_type=(jax.ShapeDtypeStruct((N_TAB, D), f32),
                  jax.ShapeDtypeStruct((N_TAB, D), f32)),
        mesh=plsc.VectorSubcoreMesh(core_axis_name="c", subcore_axis_name="s"),
        scratch_types=scratch,
    )


_dualconv_w = _make_dualconv(True)
_dualconv_u = _make_dualconv(False)


def _tc_b_body(x_ref, we_ref, wp_ref, dw0_ref, dw1_ref, dc0_ref, dc1_ref,
               tab1_ref, tab4_ref, dinvw_ref, dinvu_ref):
    degw = dw0_ref[...] + dw1_ref[...] + 1.0
    degc = dc0_ref[...] + dc1_ref[...] + 1.0
    dinvw = lax.rsqrt(degw)
    dinvu = lax.rsqrt(degc)
    dinvw_ref[...] = dinvw
    dinvu_ref[...] = dinvu
    tab1_ref[...] = dinvw * jnp.dot(x_ref[...], we_ref[...],
                                    preferred_element_type=f32)
    tab4_ref[...] = dinvu * jnp.dot(x_ref[...], wp_ref[...],
                                    preferred_element_type=f32)


def _tc_d_body(acc1_ref, acc4_ref, dinvw_ref, dinvu_ref, benc_ref, bpri_ref,
               ms_ref, pe_ref, wpm_ref, bpm_ref, wps_ref, bps_ref,
               wem_ref, wes_ref,
               tab2_ref, tab3_ref, pm_ref, ps_ref):
    enc_t = jnp.maximum(dinvw_ref[...] * acc1_ref[...] + benc_ref[...], 0.0)
    enc_t = enc_t * ms_ref[...]
    prior = jnp.maximum(dinvu_ref[...] * acc4_ref[...] + bpri_ref[...], 0.0)
    prior = prior + pe_ref[...]
    pm_ref[...] = jnp.dot(prior, wpm_ref[...],
                          preferred_element_type=f32) + bpm_ref[...]
    ps_ref[...] = jax.nn.sigmoid(
        jnp.dot(prior, wps_ref[...], preferred_element_type=f32) + bps_ref[...])
    tab2_ref[...] = dinvu_ref[...] * jnp.dot(enc_t, wem_ref[...],
                                             preferred_element_type=f32)
    tab3_ref[...] = dinvu_ref[...] * jnp.dot(enc_t, wes_ref[...],
                                             preferred_element_type=f32)


def _tc_f_body(acc2_ref, acc3_ref, dinvu_ref, bm_ref, bs_ref, pm_ref, ps_ref,
               eps_ref, rv_ref, kl_ref, cz_ref):
    enc_mean = dinvu_ref[...] * acc2_ref[...] + bm_ref[...]
    enc_std = jax.nn.sigmoid(dinvu_ref[...] * acc3_ref[...] + bs_ref[...])
    cz_ref[...] = eps_ref[...] * enc_std + enc_mean
    ps = ps_ref[...] + 1e-9
    es = enc_std + 1e-9
    kl = (2.0 * jnp.log(ps) - 2.0 * jnp.log(es)
          + (es * es + (enc_mean - pm_ref[...]) ** 2) / (ps * ps) - 1.0)
    kl_ref[0, 0] = jnp.sum(kl * rv_ref[...]) * (0.5 / N)


def kernel(edge_index, x, t, edge_score, total_len, train_len,
           W_enc, b_enc, W_enc_mean, b_enc_mean, W_enc_std, b_enc_std,
           W_prior, b_prior, W_pm, b_pm, W_ps, b_ps):
    rowvalid, zeros1 = _consts()
    maskscale, eps = _rng_consts()

    src = edge_index[0].astype(i32)
    dst = edge_index[1].astype(i32)
    pad_e = E_PAD - E
    src_p = jnp.concatenate([src, jnp.full((pad_e,), N, i32)])
    dst_p = jnp.concatenate([dst, jnp.full((pad_e,), N, i32)])
    ew_p = jnp.concatenate([edge_score.astype(f32), jnp.zeros((pad_e,), f32)])
    x_pad = jnp.concatenate([x, jnp.zeros((N_TAB - N, D), f32)])
    b_enc2 = b_enc.reshape(1, D)
    b_pri2 = b_prior.reshape(1, D)
    b_pm2 = b_pm.reshape(1, D)
    b_ps2 = b_ps.reshape(1, D)
    b_m2 = b_enc_mean.reshape(1, D)
    b_s2 = b_enc_std.reshape(1, D)
    iarr = jnp.arange(D)
    tf = jnp.asarray(t, f32)
    angle = tf / jnp.power(jnp.float32(10000.0),
                           (2.0 * (iarr // 2)).astype(f32) / D)
    pe = jnp.where(iarr % 2 == 0, jnp.sin(angle), jnp.cos(angle))
    pe = pe.astype(f32).reshape(1, D)

    degw0, degc0, degw1, degc1 = _degrees_call(dst_p, ew_p)

    tab1, tab4, dinvw, dinvu = pl.pallas_call(
        _tc_b_body,
        out_shape=(jax.ShapeDtypeStruct((N_TAB, D), f32),
                   jax.ShapeDtypeStruct((N_TAB, D), f32),
                   jax.ShapeDtypeStruct((N_TAB, 1), f32),
                   jax.ShapeDtypeStruct((N_TAB, 1), f32)),
    )(x_pad, W_enc, W_prior, degw0.reshape(N_TAB, 1),
      degw1.reshape(N_TAB, 1), degc0.reshape(N_TAB, 1),
      degc1.reshape(N_TAB, 1))

    acc1, acc4 = _dualconv_w(tab1, tab4, src_p, dst_p, ew_p)

    tab2, tab3, pm, ps = pl.pallas_call(
        _tc_d_body,
        out_shape=(jax.ShapeDtypeStruct((N_TAB, D), f32),
                   jax.ShapeDtypeStruct((N_TAB, D), f32),
                   jax.ShapeDtypeStruct((N_TAB, D), f32),
                   jax.ShapeDtypeStruct((N_TAB, D), f32)),
    )(acc1, acc4, dinvw, dinvu, b_enc2, b_pri2, maskscale, pe,
      W_pm, b_pm2, W_ps, b_ps2, W_enc_mean, W_enc_std)

    acc2, acc3 = _dualconv_u(tab2, tab3, src_p, dst_p)

    kl2d, conf_z = pl.pallas_call(
        _tc_f_body,
        out_shape=(jax.ShapeDtypeStruct((1, 1), f32),
                   jax.ShapeDtypeStruct((N_TAB, D), f32)),
        out_specs=(pl.BlockSpec(memory_space=pltpu.SMEM),
                   pl.BlockSpec(memory_space=pltpu.VMEM)),
    )(acc2, acc3, dinvu, b_m2, b_s2, pm, ps, eps, rowvalid)

    return (kl2d[0, 0], conf_z[:N])


# speedup vs baseline: 2.5089x; 2.5089x over previous
import jax
import jax.numpy as jnp
import numpy as np
from jax import lax
from jax.experimental import pallas as pl
from jax.experimental.pallas import tpu as pltpu
from jax.experimental.pallas import tpu_sc as plsc

N = 10000
E = 320000
D = 128
NC = 2
NS = 16
L = 16
N_TAB = 10112
ROWS_PER_TILE = N_TAB // NS
N_DUMMY = N_TAB - N
C = 128
E_PAD = 327680
EPT_CONV = E_PAD // NS
EPT_DEG = E_PAD // (NC * NS)

f32 = jnp.float32
i32 = jnp.int32

_CONSTS = None


def _consts():
    global _CONSTS
    if _CONSTS is None:
        rowvalid = np.zeros((N_TAB, 1), np.float32)
        rowvalid[:N] = 1.0
        zrows = np.zeros((N_TAB, D), np.float32)
        pad_idx = (N + np.arange(E_PAD - E, dtype=np.int32) % N_DUMMY)
        _CONSTS = (rowvalid, zrows, pad_idx)
    return _CONSTS


def _rng_consts():
    mask = jax.random.bernoulli(jax.random.key(42), 1.0 - 0.1, (N, D))
    maskscale = jnp.where(mask, jnp.float32(1.0 / 0.9), jnp.float32(0.0))
    maskscale = jnp.concatenate([maskscale, jnp.zeros((N_TAB - N, D), f32)])
    eps = jax.random.normal(jax.random.key(43), (N, D), dtype=f32)
    eps = jnp.concatenate([eps, jnp.zeros((N_TAB - N, D), f32)])
    return maskscale, eps


def _degrees_body(dst_hbm, ew_hbm, outw0, outc0, outw1, outc1,
                  accw_sh, accc_sh, dst_v, ew_v, ones_v, stage_v, sem):
    c = lax.axis_index("c")
    s = lax.axis_index("s")
    nbase = s * ROWS_PER_TILE
    zer = jnp.zeros((L,), f32)
    for grp in range(640 // L):
        stage_v[pl.ds(grp * L, L)] = zer
    pltpu.sync_copy(stage_v.at[pl.ds(0, ROWS_PER_TILE)],
                    accw_sh.at[pl.ds(nbase, ROWS_PER_TILE)])
    pltpu.sync_copy(stage_v.at[pl.ds(0, ROWS_PER_TILE)],
                    accc_sh.at[pl.ds(nbase, ROWS_PER_TILE)])
    one = jnp.ones((L,), f32)
    for grp in range(C // L):
        ones_v[pl.ds(grp * L, L)] = one
    plsc.subcore_barrier()

    ebase = (c * NS + s) * EPT_DEG

    def chunk(g, carry):
        off = ebase + g * C
        pltpu.sync_copy(dst_hbm.at[pl.ds(off, C)], dst_v)
        pltpu.sync_copy(ew_hbm.at[pl.ds(off, C)], ew_v)
        pltpu.sync_copy(ew_v, accw_sh.at[dst_v], add=True)
        pltpu.sync_copy(ones_v, accc_sh.at[dst_v], add=True)
        return carry

    lax.fori_loop(0, EPT_DEG // C, chunk, 0)
    plsc.subcore_barrier()

    def writeout(acc_sh, out_hbm):
        pltpu.sync_copy(acc_sh.at[pl.ds(nbase, ROWS_PER_TILE)],
                        stage_v.at[pl.ds(0, ROWS_PER_TILE)])
        pltpu.sync_copy(stage_v.at[pl.ds(0, ROWS_PER_TILE)],
                        out_hbm.at[pl.ds(nbase, ROWS_PER_TILE)])

    @pl.when(c == 0)
    def _():
        writeout(accw_sh, outw0)
        writeout(accc_sh, outc0)

    @pl.when(c == 1)
    def _():
        writeout(accw_sh, outw1)
        writeout(accc_sh, outc1)


_degrees_call = pl.kernel(
    _degrees_body,
    out_type=(jax.ShapeDtypeStruct((N_TAB,), f32),
              jax.ShapeDtypeStruct((N_TAB,), f32),
              jax.ShapeDtypeStruct((N_TAB,), f32),
              jax.ShapeDtypeStruct((N_TAB,), f32)),
    mesh=plsc.VectorSubcoreMesh(core_axis_name="c", subcore_axis_name="s"),
    scratch_types=[
        pltpu.VMEM_SHARED((N_TAB,), f32),
        pltpu.VMEM_SHARED((N_TAB,), f32),
        pltpu.VMEM((C,), i32),
        pltpu.VMEM((C,), f32),
        pltpu.VMEM((C,), f32),
        pltpu.VMEM((640,), f32),
        pltpu.SemaphoreType.DMA,
    ],
)


def _make_conv(weighted, split):
    nchunk = (EPT_DEG if split else EPT_CONV) // C
    assert nchunk % 4 == 0

    def body(tab0_hbm, tab1_hbm, src_hbm, dst_hbm, *rest):
        if weighted:
            ew_hbm = rest[0]
            rest = rest[1:]
        (out0, out1, acc_sh, src_v, dst_v, ew_v, rows_v) = rest[:7]
        sem_i = rest[7:9]
        sem_g = rest[9:11]
        sem_s = rest[11:13]
        c = lax.axis_index("c")
        s = lax.axis_index("s")
        nbase = s * ROWS_PER_TILE

        @pl.when(c == 0)
        def _():
            pltpu.sync_copy(tab0_hbm.at[pl.ds(nbase, ROWS_PER_TILE)],
                            acc_sh.at[pl.ds(nbase, ROWS_PER_TILE)])

        @pl.when(c == 1)
        def _():
            pltpu.sync_copy(tab1_hbm.at[pl.ds(nbase, ROWS_PER_TILE)],
                            acc_sh.at[pl.ds(nbase, ROWS_PER_TILE)])

        plsc.subcore_barrier()
        if split:
            ebase = c * (E_PAD // NC) + s * EPT_DEG
        else:
            ebase = s * EPT_CONV

        def issue_idx(g, b, db):
            off = ebase + g * C
            pltpu.async_copy(src_hbm.at[pl.ds(off, C)], src_v.at[b], sem_i[b])
            pltpu.async_copy(dst_hbm.at[pl.ds(off, C)], dst_v.at[db], sem_i[b])
            if weighted:
                @pl.when(c == 0)
                def _():
                    pltpu.async_copy(ew_hbm.at[pl.ds(off, C)], ew_v.at[b],
                                     sem_i[b])

        def wait_idx(b, db):
            pltpu.make_async_copy(src_hbm.at[pl.ds(0, C)], src_v.at[b],
                                  sem_i[b]).wait()
            pltpu.make_async_copy(dst_hbm.at[pl.ds(0, C)], dst_v.at[db],
                                  sem_i[b]).wait()
            if weighted:
                @pl.when(c == 0)
                def _():
                    pltpu.make_async_copy(src_hbm.at[pl.ds(0, C)],
                                          ew_v.at[b], sem_i[b]).wait()

        def issue_gather(b):
            if split:
                pltpu.async_copy(tab0_hbm.at[src_v.at[b]], rows_v.at[b],
                                 sem_g[b])
            else:
                @pl.when(c == 0)
                def _():
                    pltpu.async_copy(tab0_hbm.at[src_v.at[b]], rows_v.at[b],
                                     sem_g[b])

                @pl.when(c == 1)
                def _():
                    pltpu.async_copy(tab1_hbm.at[src_v.at[b]], rows_v.at[b],
                                     sem_g[b])

        def wait_gather(b):
            pltpu.make_async_copy(tab0_hbm.at[src_v.at[b]], rows_v.at[b],
                                  sem_g[b]).wait()

        def mult(b):
            if not weighted:
                return

            @pl.when(c == 0)
            def _():
                rb = rows_v.at[b]
                eb = ew_v.at[b]

                def mult_group(i, cc):
                    ewv = eb[pl.ds(i * L, L)]
                    for l in range(L):
                        sv = ewv.at[jnp.full((L,), l, i32)].get(
                            mode="promise_in_bounds")
                        e = i * L + l
                        for j in range(D // L):
                            rb[e, pl.ds(j * L, L)] = (
                                rb[e, pl.ds(j * L, L)] * sv)
                    return cc

                lax.fori_loop(0, C // L, mult_group, 0)

        def issue_scatter(b, db):
            pltpu.async_copy(rows_v.at[b], acc_sh.at[dst_v.at[db]],
                             sem_s[b], add=True)

        def wait_scatter(b, db):
            pltpu.make_async_copy(rows_v.at[b], acc_sh.at[dst_v.at[db]],
                                  sem_s[b]).wait()

        def iter_body(g, b, db, nxt_idx, wait_prev_sc, nxt_gather):
            wait_gather(b)
            mult(b)
            issue_scatter(b, db)
            if nxt_idx:
                issue_idx(g + 2, b, (db + 2) % 4)
            if nxt_gather:
                b2 = 1 - b
                wait_idx(b2, (db + 1) % 4)
                if wait_prev_sc:
                    wait_scatter(b2, (db + 3) % 4)
                issue_gather(b2)

        issue_idx(0, 0, 0)
        issue_idx(1, 1, 1)
        wait_idx(0, 0)
        issue_gather(0)
        iter_body(0, 0, 0, True, False, True)
        iter_body(1, 1, 1, True, True, True)

        def block(g0, carry):
            base = 2 + g0 * 4
            for k in range(4):
                iter_body(base + k, k % 2, (2 + k) % 4, True, True, True)
            return carry

        lax.fori_loop(0, (nchunk - 4) // 4, block, 0)
        iter_body(nchunk - 2, 0, 2, False, True, True)
        iter_body(nchunk - 1, 1, 3, False, False, False)
        wait_scatter(0, 2)
        wait_scatter(1, 3)
        plsc.subcore_barrier()

        @pl.when(c == 0)
        def _():
            pltpu.sync_copy(acc_sh.at[pl.ds(nbase, ROWS_PER_TILE)],
                            out0.at[pl.ds(nbase, ROWS_PER_TILE)])

        @pl.when(c == 1)
        def _():
            pltpu.sync_copy(acc_sh.at[pl.ds(nbase, ROWS_PER_TILE)],
                            out1.at[pl.ds(nbase, ROWS_PER_TILE)])

    scratch = [
        pltpu.VMEM_SHARED((N_TAB, D), f32),
        pltpu.VMEM((2, C), i32),
        pltpu.VMEM((4, C), i32),
        pltpu.VMEM((2, C), f32),
        pltpu.VMEM((2, C, D), f32),
    ] + [pltpu.SemaphoreType.DMA] * 6
    return pl.kernel(
        body,
        out_type=(jax.ShapeDtypeStruct((N_TAB, D), f32),
                  jax.ShapeDtypeStruct((N_TAB, D), f32)),
        mesh=plsc.VectorSubcoreMesh(core_axis_name="c", subcore_axis_name="s"),
        scratch_types=scratch,
    )


_conv_a = _make_conv(weighted=True, split=False)
_conv_b = _make_conv(weighted=False, split=True)


def _tc_b_body(x_ref, dw0_ref, dw1_ref, dc0_ref, dc1_ref,
               r1_ref, r4_ref, dinvw_ref, dinvu_ref):
    degw = dw0_ref[...] + dw1_ref[...] + 1.0
    degc = dc0_ref[...] + dc1_ref[...] + 1.0
    dinvw = lax.rsqrt(degw)
    dinvu = lax.rsqrt(degc)
    dinvw_ref[...] = dinvw
    dinvu_ref[...] = dinvu
    r1_ref[...] = dinvw * x_ref[...]
    r4_ref[...] = dinvu * x_ref[...]


def _tc_d_body(acc1_ref, acc4_ref, dinvw_ref, dinvu_ref, we_ref, wp_ref,
               benc_ref, bpri_ref, ms_ref, pe_ref, wpm_ref, bpm_ref,
               wps_ref, bps_ref,
               q_ref, pm_ref, ps_ref):
    enc_t = jnp.maximum(
        dinvw_ref[...] * jnp.dot(acc1_ref[...], we_ref[...],
                                 preferred_element_type=f32) + benc_ref[...],
        0.0) * ms_ref[...]
    prior = jnp.maximum(
        dinvu_ref[...] * jnp.dot(acc4_ref[...], wp_ref[...],
                                 preferred_element_type=f32) + bpri_ref[...],
        0.0) + pe_ref[...]
    pm_ref[...] = jnp.dot(prior, wpm_ref[...],
                          preferred_element_type=f32) + bpm_ref[...]
    ps_ref[...] = jax.nn.sigmoid(
        jnp.dot(prior, wps_ref[...], preferred_element_type=f32) + bps_ref[...])
    q_ref[...] = dinvu_ref[...] * enc_t


def _tc_f_body(sb0_ref, sb1_ref, dinvu_ref, wm_ref, bm_ref, ws_ref, bs_ref,
               pm_ref, ps_ref, eps_ref, rv_ref, kl_ref, cz_ref):
    s_sum = sb0_ref[...] + sb1_ref[...]
    enc_mean = dinvu_ref[...] * jnp.dot(s_sum, wm_ref[...],
                                        preferred_element_type=f32) + bm_ref[...]
    enc_std = jax.nn.sigmoid(
        dinvu_ref[...] * jnp.dot(s_sum, ws_ref[...],
                                 preferred_element_type=f32) + bs_ref[...])
    cz_ref[...] = eps_ref[...] * enc_std + enc_mean
    ps = ps_ref[...] + 1e-9
    es = enc_std + 1e-9
    kl = (2.0 * jnp.log(ps) - 2.0 * jnp.log(es)
          + (es * es + (enc_mean - pm_ref[...]) ** 2) / (ps * ps) - 1.0)
    kl_ref[0, 0] = jnp.sum(kl * rv_ref[...]) * (0.5 / N)


def kernel(edge_index, x, t, edge_score, total_len, train_len,
           W_enc, b_enc, W_enc_mean, b_enc_mean, W_enc_std, b_enc_std,
           W_prior, b_prior, W_pm, b_pm, W_ps, b_ps):
    rowvalid, zrows, pad_idx = _consts()
    maskscale, eps = _rng_consts()

    src = edge_index[0].astype(i32)
    dst = edge_index[1].astype(i32)
    src_p = jnp.concatenate([src, jnp.asarray(pad_idx)])
    dst_p = jnp.concatenate([dst, jnp.asarray(pad_idx)])
    ew_p = jnp.concatenate([edge_score.astype(f32),
                            jnp.zeros((E_PAD - E,), f32)])
    x_pad = jnp.concatenate([x, jnp.zeros((N_TAB - N, D), f32)])
    b_enc2 = b_enc.reshape(1, D)
    b_pri2 = b_prior.reshape(1, D)
    b_pm2 = b_pm.reshape(1, D)
    b_ps2 = b_ps.reshape(1, D)
    b_m2 = b_enc_mean.reshape(1, D)
    b_s2 = b_enc_std.reshape(1, D)
    iarr = jnp.arange(D)
    tf = jnp.asarray(t, f32)
    angle = tf / jnp.power(jnp.float32(10000.0),
                           (2.0 * (iarr // 2)).astype(f32) / D)
    pe = jnp.where(iarr % 2 == 0, jnp.sin(angle), jnp.cos(angle))
    pe = pe.astype(f32).reshape(1, D)

    degw0, degc0, degw1, degc1 = _degrees_call(dst_p, ew_p)

    r1, r4, dinvw, dinvu = pl.pallas_call(
        _tc_b_body,
        out_shape=(jax.ShapeDtypeStruct((N_TAB, D), f32),
                   jax.ShapeDtypeStruct((N_TAB, D), f32),
                   jax.ShapeDtypeStruct((N_TAB, 1), f32),
                   jax.ShapeDtypeStruct((N_TAB, 1), f32)),
    )(x_pad, degw0.reshape(N_TAB, 1), degw1.reshape(N_TAB, 1),
      degc0.reshape(N_TAB, 1), degc1.reshape(N_TAB, 1))

    acc1, acc4 = _conv_a(r1, r4, src_p, dst_p, ew_p)

    q, pm, ps = pl.pallas_call(
        _tc_d_body,
        out_shape=(jax.ShapeDtypeStruct((N_TAB, D), f32),
                   jax.ShapeDtypeStruct((N_TAB, D), f32),
                   jax.ShapeDtypeStruct((N_TAB, D), f32)),
    )(acc1, acc4, dinvw, dinvu, W_enc, W_prior, b_enc2, b_pri2, maskscale,
      pe, W_pm, b_pm2, W_ps, b_ps2)

    sb0, sb1 = _conv_b(q, zrows, src_p, dst_p)

    kl2d, conf_z = pl.pallas_call(
        _tc_f_body,
        out_shape=(jax.ShapeDtypeStruct((1, 1), f32),
                   jax.ShapeDtypeStruct((N_TAB, D), f32)),
        out_specs=(pl.BlockSpec(memory_space=pltpu.SMEM),
                   pl.BlockSpec(memory_space=pltpu.VMEM)),
    )(sb0, sb1, dinvu, W_enc_mean, b_m2, W_enc_std, b_s2, pm, ps, eps,
      rowvalid)

    return (kl2d[0, 0], conf_z[:N])
